# baseline (speedup 1.0000x reference)
import functools

import jax
import jax.numpy as jnp
from jax import lax
from jax.experimental import pallas as pl
from jax.experimental.pallas import tpu as pltpu
from jax.experimental.pallas import tpu_sc as plsc

NUM_EMBEDDINGS = 1000000
EMBEDDING_DIM = 64
BATCH = 4096 * 200

NUM_CORES = 2
NUM_SUBCORES = 16
NUM_WORKERS = NUM_CORES * NUM_SUBCORES

G = 128
ROWS_PER_WORKER = BATCH // NUM_WORKERS
NG = ROWS_PER_WORKER // G
NG_TOTAL = BATCH // G

_mesh = plsc.VectorSubcoreMesh(core_axis_name="c", subcore_axis_name="s")


@functools.partial(
    pl.kernel,
    out_type=jax.ShapeDtypeStruct((BATCH, EMBEDDING_DIM), jnp.float32),
    mesh=_mesh,
    compiler_params=pltpu.CompilerParams(use_tc_tiling_on_sc=False),
    scratch_types=[
        pltpu.VMEM((NG, G), jnp.int32),
        pltpu.VMEM((G, EMBEDDING_DIM), jnp.float32),
        pltpu.SemaphoreType.DMA,
    ],
)
def _embed_sc(table_hbm, idx_hbm, out_hbm, idx_v, rows_v, sem):
    wid = lax.axis_index("s") * NUM_CORES + lax.axis_index("c")
    base = wid * ROWS_PER_WORKER
    pltpu.sync_copy(idx_hbm.at[pl.ds(wid * NG, NG)], idx_v)

    @pl.loop(0, NG)
    def _chunk(j):
        pltpu.async_copy(table_hbm.at[idx_v.at[j]], rows_v, sem).wait()
        pltpu.sync_copy(rows_v, out_hbm.at[pl.ds(base + j * G, G)])


def kernel(token_ids, weight):
    idx = token_ids.astype(jnp.int32).reshape(NG_TOTAL, G)
    out = _embed_sc(weight, idx)
    return out.reshape(token_ids.shape[0], token_ids.shape[1], EMBEDDING_DIM)


# speedup vs baseline: 1.1162x; 1.1162x over previous
import functools

import jax
import jax.numpy as jnp
from jax import lax
from jax.experimental import pallas as pl
from jax.experimental.pallas import tpu as pltpu
from jax.experimental.pallas import tpu_sc as plsc

NUM_EMBEDDINGS = 1000000
EMBEDDING_DIM = 64
BATCH = 4096 * 200

NUM_CORES = 2
NUM_SUBCORES = 16
NUM_WORKERS = NUM_CORES * NUM_SUBCORES

G = 128
KG = 4
ROWS_PER_WORKER = BATCH // NUM_WORKERS
NG = ROWS_PER_WORKER // G
NGRP = NG // KG
GROUP_ROWS = KG * G
NG_TOTAL = BATCH // G

_mesh = plsc.VectorSubcoreMesh(core_axis_name="c", subcore_axis_name="s")


@functools.partial(
    pl.kernel,
    out_type=jax.ShapeDtypeStruct((BATCH, EMBEDDING_DIM), jnp.float32),
    mesh=_mesh,
    compiler_params=pltpu.CompilerParams(use_tc_tiling_on_sc=False),
    scratch_types=[
        pltpu.VMEM((NG, G), jnp.int32),
        pltpu.VMEM((2, GROUP_ROWS, EMBEDDING_DIM), jnp.float32),
        pltpu.SemaphoreType.DMA,
        pltpu.SemaphoreType.DMA,
    ],
)
def _embed_sc(table_hbm, idx_hbm, out_hbm, idx_v, rows_v, sem_g, sem_s):
    wid = lax.axis_index("s") * NUM_CORES + lax.axis_index("c")
    base = wid * ROWS_PER_WORKER
    pltpu.sync_copy(idx_hbm.at[pl.ds(wid * NG, NG)], idx_v)

    def fire_gathers(grp, buf):
        for t in range(KG):
            pltpu.async_copy(
                table_hbm.at[idx_v.at[grp * KG + t]],
                rows_v.at[buf, pl.ds(t * G, G)],
                sem_g,
            )

    fire_gathers(0, 0)

    @pl.loop(0, NGRP)
    def _group(g):
        buf = lax.rem(g, 2)

        @pl.when(g > 0)
        def _():
            pltpu.make_async_copy(
                rows_v.at[1 - buf], out_hbm.at[pl.ds(0, GROUP_ROWS)], sem_s
            ).wait()

        @pl.when(g + 1 < NGRP)
        def _():
            fire_gathers(g + 1, 1 - buf)

        for t in range(KG):
            pltpu.make_async_copy(
                table_hbm.at[idx_v.at[0]], rows_v.at[buf, pl.ds(0, G)], sem_g
            ).wait()
        pltpu.async_copy(
            rows_v.at[buf], out_hbm.at[pl.ds(base + g * GROUP_ROWS, GROUP_ROWS)], sem_s
        )

    pltpu.make_async_copy(
        rows_v.at[(NGRP - 1) % 2], out_hbm.at[pl.ds(0, GROUP_ROWS)], sem_s
    ).wait()


def kernel(token_ids, weight):
    idx = token_ids.astype(jnp.int32).reshape(NG_TOTAL, G)
    out = _embed_sc(weight, idx)
    return out.reshape(token_ids.shape[0], token_ids.shape[1], EMBEDDING_DIM)
